# baseline (speedup 1.0000x reference)
import functools

import jax
import jax.numpy as jnp
from jax import lax
from jax.experimental import pallas as pl
from jax.experimental.pallas import tpu as pltpu
from jax.experimental.pallas import tpu_sc as plsc

B = 4096
S = 200
D = 64
NC = 2
NS = 16
NW = NC * NS
BPW = B // NW

_MESH = dict(core_axis_name="c", subcore_axis_name="s")
_SC_PARAMS = pltpu.CompilerParams(
    use_tc_tiling_on_sc=False, needs_layout_passes=False)

_SPLIT = (0, 104), (104, 96)


def _sc_gather_pair(e_idx, r_idx, We, Wr):
    @functools.partial(
        pl.kernel,
        out_type=(
            jax.ShapeDtypeStruct((B, D), jnp.float32),
            jax.ShapeDtypeStruct((B, D), jnp.float32),
        ),
        mesh=plsc.VectorSubcoreMesh(**_MESH),
        compiler_params=_SC_PARAMS,
        scratch_types=[
            pltpu.VMEM((BPW,), jnp.int32),
            pltpu.VMEM((BPW, D), jnp.float32),
            pltpu.SemaphoreType.DMA,
        ],
    )
    def k(e_hbm, r_hbm, we_hbm, wr_hbm, e_out, r_out, idx_v, rows_v, sem):
        wid = lax.axis_index("s") * NC + lax.axis_index("c")
        base = wid * BPW
        pltpu.sync_copy(e_hbm.at[pl.ds(base, BPW)], idx_v)
        pltpu.async_copy(we_hbm.at[idx_v], rows_v, sem).wait()
        pltpu.sync_copy(rows_v, e_out.at[pl.ds(base, BPW)])
        pltpu.sync_copy(r_hbm.at[pl.ds(base, BPW)], idx_v)
        pltpu.async_copy(wr_hbm.at[idx_v], rows_v, sem).wait()
        pltpu.sync_copy(rows_v, r_out.at[pl.ds(base, BPW)])

    return k(e_idx, r_idx, We, Wr)


def _tc_comb(et, e_emb, r_emb, Deh, Drh, Det, Drt, bc):
    def body(et_ref, e_ref, r_ref, deh, drh, det, drt, bc_ref, o_ref):
        dn = (((1,), (1,)), ((), ()))
        e = e_ref[...]
        r = r_ref[...]
        ch = lax.dot_general(e, deh[...], dn, preferred_element_type=jnp.float32)
        ch = ch + lax.dot_general(r, drh[...], dn, preferred_element_type=jnp.float32)
        ct = lax.dot_general(e, det[...], dn, preferred_element_type=jnp.float32)
        ct = ct + lax.dot_general(r, drt[...], dn, preferred_element_type=jnp.float32)
        c = jnp.where(et_ref[0] == 0, ch, ct) + bc_ref[...]
        o_ref[...] = jnp.tanh(c)

    vmem = pl.BlockSpec(memory_space=pltpu.VMEM)
    return pl.pallas_call(
        body,
        out_shape=jax.ShapeDtypeStruct((B, D), jnp.float32),
        in_specs=[pl.BlockSpec(memory_space=pltpu.SMEM)] + [vmem] * 7,
        out_specs=vmem,
    )(et, e_emb, r_emb, Deh, Drh, Det, Drt, bc)


def _sc_score(samples, We, u, bp16):
    NBUF = 4

    @functools.partial(
        pl.kernel,
        out_type=jax.ShapeDtypeStruct((B, S), jnp.float32),
        mesh=plsc.VectorSubcoreMesh(**_MESH),
        compiler_params=_SC_PARAMS,
        scratch_types=[
            pltpu.VMEM((BPW, S), jnp.int32),
            pltpu.VMEM((BPW, D), jnp.float32),
            pltpu.VMEM((BPW, S), jnp.float32),
            pltpu.VMEM((16,), jnp.float32),
            [pltpu.VMEM((S, D), jnp.float32) for _ in range(NBUF)],
            [pltpu.SemaphoreType.DMA for _ in range(NBUF)],
        ],
    )
    def k(samples_hbm, we_hbm, u_hbm, bp_hbm, out_hbm,
          samples_v, u_v, out_v, bp_v, bufs, sems):
        wid = lax.axis_index("s") * NC + lax.axis_index("c")
        base = wid * BPW
        pltpu.sync_copy(samples_hbm.at[pl.ds(base, BPW), :], samples_v)
        pltpu.sync_copy(u_hbm.at[pl.ds(base, BPW), :], u_v)
        pltpu.sync_copy(bp_hbm, bp_v)

        def gather(b, buf, sem):
            for off, n in _SPLIT:
                yield pltpu.make_async_copy(
                    we_hbm.at[samples_v.at[b, pl.ds(off, n)]],
                    buf.at[pl.ds(off, n)], sem)

        def issue(b, buf, sem):
            for cp in gather(b, buf, sem):
                cp.start()

        def drain(b, buf, sem):
            for cp in gather(b, buf, sem):
                cp.wait()

        lane = lax.iota(jnp.int32, 16)
        bpv = bp_v[...]

        def compute(b, buf):
            u0 = u_v[b, pl.ds(0, 16)]
            u1 = u_v[b, pl.ds(16, 16)]
            u2 = u_v[b, pl.ds(32, 16)]
            u3 = u_v[b, pl.ds(48, 16)]

            def dot16(r0):
                acc = jnp.zeros((16,), jnp.float32)
                for rr in range(16):
                    rw = r0 + rr
                    t = buf[rw, pl.ds(0, 16)] * u0
                    t = t + buf[rw, pl.ds(16, 16)] * u1
                    t = t + buf[rw, pl.ds(32, 16)] * u2
                    t = t + buf[rw, pl.ds(48, 16)] * u3
                    s = jnp.sum(t)
                    acc = jnp.where(lane == rr, s, acc)
                return acc

            def sig16(r0):
                acc = dot16(r0)
                return 1.0 / (1.0 + jnp.exp(-(acc + bpv)))

            def g_body(g, carry):
                out_v[b, pl.ds(g * 16, 16)] = sig16(g * 16)
                return carry

            lax.fori_loop(0, (S // 16), g_body, 0)
            out_v[b, pl.ds(S - 16, 16)] = sig16(S - 16)

        for j in range(NBUF):
            issue(j, bufs[j], sems[j])

        def b_body(i, carry):
            for j in range(NBUF):
                b = i * NBUF + j
                drain(b, bufs[j], sems[j])
                compute(b, bufs[j])

                @pl.when(b + NBUF < BPW)
                def _():
                    issue(b + NBUF, bufs[j], sems[j])
            return carry

        lax.fori_loop(0, BPW // NBUF, b_body, 0)
        pltpu.sync_copy(out_v, out_hbm.at[pl.ds(base, BPW), :])

    return k(samples, We, u, bp16)


def kernel(e, r, samples, entity_type, We, Wr, Deh, Drh, Det, Drt, bc, bp):
    e = e.astype(jnp.int32)
    r = r.astype(jnp.int32)
    samples = samples.astype(jnp.int32)
    et = jnp.asarray(entity_type, jnp.int32).reshape(1)
    e_emb, r_emb = _sc_gather_pair(e, r, We, Wr)
    u = _tc_comb(et, e_emb, r_emb, Deh, Drh, Det, Drt, bc.reshape(1, D))
    bp16 = jnp.broadcast_to(bp.astype(jnp.float32), (16,))
    return _sc_score(samples, We, u, bp16)
